# baseline (speedup 1.0000x reference)
import jax
import jax.numpy as jnp
from jax import lax
from jax.experimental import pallas as pl
from jax.experimental.pallas import tpu as pltpu

_B = 2
_C = 128
_DH = 32
_HWD = 16
_N = _HWD * _HWD * _HWD
_PAD = 32
_PW = _N + 2 * _PAD
_HPAD = 256
_VW = _N + 2 * _HPAD


def _fused_block(x_ref, w3_ref, wg_ref, bg_ref, wpair_ref, w2_ref, gnw_ref,
                 gnb_ref, out_ref, vstk_ref):
    f32 = jnp.float32
    bf16 = jnp.bfloat16
    bs = range(_B)
    X = [x_ref[b] for b in bs]
    Xb = [X[b].astype(bf16) for b in bs]

    qkvf = [jnp.dot(w3_ref[...], Xb[b], preferred_element_type=f32)
            for b in bs]
    qkv = [qkvf[b].astype(bf16) for b in bs]
    qT = [qkv[b][0:_C, :] for b in bs]
    kT = [qkv[b][_C:2 * _C, :] for b in bs]
    vT = [qkv[b][2 * _C:3 * _C, :] for b in bs]

    dotT = lambda a, c: lax.dot_general(a, c, (((1,), (1,)), ((), ())),
                                        preferred_element_type=f32)
    Sraw = [dotT(qT[b], kT[b]) for b in bs]
    qksq = [jnp.sum(qkvf[b][0:2 * _C, :] * qkvf[b][0:2 * _C, :],
                    axis=1, keepdims=True) for b in bs]
    rq = [1.0 / jnp.maximum(jnp.sqrt(qksq[b][0:_C]), 1e-12) for b in bs]
    rk = [1.0 / jnp.maximum(jnp.sqrt(qksq[b][_C:2 * _C]), 1e-12).reshape(1, _C)
          for b in bs]

    r1 = lax.broadcasted_iota(jnp.int32, (_C, _C), 0) // _DH
    r2 = lax.broadcasted_iota(jnp.int32, (_C, _C), 1) // _DH
    blk = r1 == r2
    A = []
    for b in bs:
        S = jnp.where(blk, Sraw[b] * rq[b] * rk[b], -1e30)
        S = S - jnp.max(S, axis=1, keepdims=True)
        E = jnp.exp(S)
        A.append(E / jnp.sum(E, axis=1, keepdims=True))

    WgA = [jnp.dot(wg_ref[...], A[b].astype(bf16),
                   preferred_element_type=f32).astype(bf16) for b in bs]
    CAt = [jnp.dot(WgA[b], vT[b], preferred_element_type=f32) for b in bs]
    attn = [X[b] + CAt[b] + bg_ref[...] for b in bs]

    n = lax.broadcasted_iota(jnp.int32, (1, _N), 1)
    ww = (n >> 4) & 15
    dd = n & 15
    for b in bs:
        vstk_ref[b, :, 0:_HPAD] = jnp.zeros((9 * _C, _HPAD), bf16)
        vstk_ref[b, :, _HPAD + _N:_VW] = jnp.zeros((9 * _C, _HPAD), bf16)
        vstk_ref[b, 4 * _C:5 * _C, _HPAD:_HPAD + _N] = attn[b].astype(bf16)

    for b in bs:
        j = 0
        for db in (-1, 0, 1):
            for dc in (-1, 0, 1):
                if db == 0 and dc == 0:
                    j += 1
                    continue
                s = db * 16 + dc
                sl = vstk_ref[b, 4 * _C:5 * _C, _HPAD + s:_HPAD + s + _N]
                cond = None
                for d, v in ((db, ww), (dc, dd)):
                    if d == -1:
                        c = v >= 1
                    elif d == 1:
                        c = v <= 14
                    else:
                        continue
                    cond = c if cond is None else (cond & c)
                if cond is not None:
                    sl = jnp.where(cond, sl, jnp.zeros((), bf16))
                vstk_ref[b, j * _C:(j + 1) * _C, _HPAD:_HPAD + _N] = sl
                j += 1

    _T = _N // 2
    P01 = [jnp.dot(wpair_ref[...], vstk_ref[b], preferred_element_type=f32)
           for b in bs]
    P2 = [jnp.dot(w2_ref[...], vstk_ref[b, :, 512:512 + _N],
                  preferred_element_type=f32) for b in bs]
    acc = [[None, None] for _ in bs]
    s1 = [None, None]
    s2 = [None, None]
    for b in bs:
        for t in range(2):
            c0 = t * _T
            a = (P01[b][0:_C, c0:c0 + _T]
                 + P01[b][_C:2 * _C, 256 + c0:256 + c0 + _T]
                 + P2[b][:, c0:c0 + _T])
            acc[b][t] = a
            p1 = jnp.sum(a)
            p2 = jnp.sum(a * a)
            s1[b] = p1 if t == 0 else s1[b] + p1
            s2[b] = p2 if t == 0 else s2[b] + p2

    cnt = float(_C * _N)
    for b in bs:
        mean = s1[b] / cnt
        var = s2[b] / cnt - mean * mean
        rstd = lax.rsqrt(var + 1e-5)
        for t in range(2):
            c0 = t * _T
            gn = (acc[b][t] - mean) * rstd * gnw_ref[...] + gnb_ref[...]
            o = gn + attn[b][:, c0:c0 + _T]
            out_ref[b, :, c0:c0 + _T] = jnp.where(o >= 0, o, 0.01 * o)


def kernel(x, Wqkvv, Wout1, bout1, Wout2, bout2, gamma1, conv_w, gn_w, gn_b,
           index_sample):
    B, C, H, W, D = x.shape
    N = H * W * D
    x_cn = x.reshape(B, C, N)
    w3 = Wqkvv[:3 * C, :].astype(jnp.bfloat16)
    wg = (gamma1[:, None] * Wout2).astype(jnp.bfloat16)
    bg = (gamma1 * bout2).reshape(C, 1)
    wstk = conv_w.astype(jnp.bfloat16).transpose(2, 0, 3, 4, 1).reshape(3, C, 9 * C)
    wpair = wstk[0:2].reshape(2 * C, 9 * C)
    w2 = wstk[2]

    full = lambda *shape: pl.BlockSpec(shape, lambda: (0,) * len(shape))
    out = pl.pallas_call(
        _fused_block,
        grid=(),
        in_specs=[
            full(B, C, N),
            full(3 * C, C),
            full(C, C),
            full(C, 1),
            full(2 * C, 9 * C),
            full(C, 9 * C),
            full(C, 1),
            full(C, 1),
        ],
        out_specs=full(B, C, N),
        out_shape=jax.ShapeDtypeStruct((B, C, N), jnp.float32),
        scratch_shapes=[pltpu.VMEM((B, 9 * C, _VW), jnp.bfloat16)],
    )(x_cn, w3, wg, bg, wpair, w2, gn_w.reshape(C, 1), gn_b.reshape(C, 1))
    return out.reshape(B, C, H, W, D)


# speedup vs baseline: 1.1351x; 1.1351x over previous
import jax
import jax.numpy as jnp
from jax import lax
from jax.experimental import pallas as pl
from jax.experimental.pallas import tpu as pltpu

_B = 2
_C = 128
_DH = 32
_HWD = 16
_N = _HWD * _HWD * _HWD
_PAD = 32
_PW = _N + 2 * _PAD
_HPAD = 256
_VW = _N + 2 * _HPAD


def _fused_block(x_ref, w3_ref, wg_ref, bg_ref, wpair_ref, w2_ref, gnw_ref,
                 gnb_ref, out_ref, vstk_ref):
    f32 = jnp.float32
    bf16 = jnp.bfloat16
    bs = range(_B)
    X = [x_ref[b] for b in bs]
    Xb = [X[b].astype(bf16) for b in bs]

    qkvf = [jnp.dot(w3_ref[...], Xb[b], preferred_element_type=f32)
            for b in bs]
    qkv = [qkvf[b].astype(bf16) for b in bs]
    qT = [qkv[b][0:_C, :] for b in bs]
    kT = [qkv[b][_C:2 * _C, :] for b in bs]
    vT = [qkv[b][2 * _C:3 * _C, :] for b in bs]

    dotT = lambda a, c: lax.dot_general(a, c, (((1,), (1,)), ((), ())),
                                        preferred_element_type=f32)
    Sraw = [dotT(qT[b], kT[b]) for b in bs]
    qksq = [jnp.sum(qkvf[b][0:2 * _C, :] * qkvf[b][0:2 * _C, :],
                    axis=1, keepdims=True) for b in bs]
    rq = [1.0 / jnp.maximum(jnp.sqrt(qksq[b][0:_C]), 1e-12) for b in bs]
    rk = [1.0 / jnp.maximum(jnp.sqrt(qksq[b][_C:2 * _C]), 1e-12).reshape(1, _C)
          for b in bs]

    r1 = lax.broadcasted_iota(jnp.int32, (_C, _C), 0) // _DH
    r2 = lax.broadcasted_iota(jnp.int32, (_C, _C), 1) // _DH
    blk = r1 == r2
    A = []
    for b in bs:
        S = jnp.where(blk, Sraw[b] * rq[b] * rk[b], -1e30)
        S = S - jnp.max(S, axis=1, keepdims=True)
        E = jnp.exp(S)
        A.append(E / jnp.sum(E, axis=1, keepdims=True))

    WgA = [jnp.dot(wg_ref[...], A[b].astype(bf16),
                   preferred_element_type=f32).astype(bf16) for b in bs]
    CAt = [jnp.dot(WgA[b], vT[b], preferred_element_type=f32) for b in bs]
    attn = [X[b] + CAt[b] + bg_ref[...] for b in bs]

    n = lax.broadcasted_iota(jnp.int32, (1, _N), 1)
    ww = (n >> 4) & 15
    dd = n & 15
    for b in bs:
        vstk_ref[b, :, 0:_HPAD] = jnp.zeros((9 * _C, _HPAD), bf16)
        vstk_ref[b, :, _HPAD + _N:_VW] = jnp.zeros((9 * _C, _HPAD), bf16)
        vstk_ref[b, 4 * _C:5 * _C, _HPAD:_HPAD + _N] = attn[b].astype(bf16)

    for b in bs:
        j = 0
        for db in (-1, 0, 1):
            for dc in (-1, 0, 1):
                if db == 0 and dc == 0:
                    j += 1
                    continue
                s = db * 16 + dc
                sl = vstk_ref[b, 4 * _C:5 * _C, _HPAD + s:_HPAD + s + _N]
                cond = None
                for d, v in ((db, ww), (dc, dd)):
                    if d == -1:
                        c = v >= 1
                    elif d == 1:
                        c = v <= 14
                    else:
                        continue
                    cond = c if cond is None else (cond & c)
                if cond is not None:
                    sl = jnp.where(cond, sl, jnp.zeros((), bf16))
                vstk_ref[b, j * _C:(j + 1) * _C, _HPAD:_HPAD + _N] = sl
                j += 1

    _T = _N // 2
    acc = [[None, None] for _ in bs]
    s1 = [None, None]
    s2 = [None, None]
    for b in bs:
        for t in range(2):
            c0 = t * _T
            a = (jnp.dot(wpair_ref[0:_C], vstk_ref[b, :, c0:c0 + _T],
                         preferred_element_type=f32)
                 + jnp.dot(wpair_ref[_C:2 * _C],
                           vstk_ref[b, :, 256 + c0:256 + c0 + _T],
                           preferred_element_type=f32)
                 + jnp.dot(w2_ref[...], vstk_ref[b, :, 512 + c0:512 + c0 + _T],
                           preferred_element_type=f32))
            acc[b][t] = a
            p1 = jnp.sum(a)
            p2 = jnp.sum(a * a)
            s1[b] = p1 if t == 0 else s1[b] + p1
            s2[b] = p2 if t == 0 else s2[b] + p2

    cnt = float(_C * _N)
    for b in bs:
        mean = s1[b] / cnt
        var = s2[b] / cnt - mean * mean
        rstd = lax.rsqrt(var + 1e-5)
        for t in range(2):
            c0 = t * _T
            gn = (acc[b][t] - mean) * rstd * gnw_ref[...] + gnb_ref[...]
            o = gn + attn[b][:, c0:c0 + _T]
            out_ref[b, :, c0:c0 + _T] = jnp.where(o >= 0, o, 0.01 * o)


def kernel(x, Wqkvv, Wout1, bout1, Wout2, bout2, gamma1, conv_w, gn_w, gn_b,
           index_sample):
    B, C, H, W, D = x.shape
    N = H * W * D
    x_cn = x.reshape(B, C, N)
    w3 = Wqkvv[:3 * C, :].astype(jnp.bfloat16)
    wg = (gamma1[:, None] * Wout2).astype(jnp.bfloat16)
    bg = (gamma1 * bout2).reshape(C, 1)
    wstk = conv_w.astype(jnp.bfloat16).transpose(2, 0, 3, 4, 1).reshape(3, C, 9 * C)
    wpair = wstk[0:2].reshape(2 * C, 9 * C)
    w2 = wstk[2]

    full = lambda *shape: pl.BlockSpec(shape, lambda: (0,) * len(shape))
    out = pl.pallas_call(
        _fused_block,
        grid=(),
        in_specs=[
            full(B, C, N),
            full(3 * C, C),
            full(C, C),
            full(C, 1),
            full(2 * C, 9 * C),
            full(C, 9 * C),
            full(C, 1),
            full(C, 1),
        ],
        out_specs=full(B, C, N),
        out_shape=jax.ShapeDtypeStruct((B, C, N), jnp.float32),
        scratch_shapes=[pltpu.VMEM((B, 9 * C, _VW), jnp.bfloat16)],
    )(x_cn, w3, wg, bg, wpair, w2, gn_w.reshape(C, 1), gn_b.reshape(C, 1))
    return out.reshape(B, C, H, W, D)


# speedup vs baseline: 1.2189x; 1.0738x over previous
import jax
import jax.numpy as jnp
from jax import lax
from jax.experimental import pallas as pl
from jax.experimental.pallas import tpu as pltpu

_B = 2
_C = 128
_DH = 32
_HWD = 16
_N = _HWD * _HWD * _HWD
_PAD = 32
_PW = _N + 2 * _PAD
_HPAD = 256
_VW = _N + 2 * _HPAD


def _fused_block(x_ref, w3_ref, wg_ref, bg_ref, wstk_ref, gnw_ref,
                 gnb_ref, out_ref, vstk_ref):
    f32 = jnp.float32
    bf16 = jnp.bfloat16
    bs = range(_B)
    X = [x_ref[b] for b in bs]
    Xb = [X[b].astype(bf16) for b in bs]

    qkvf = [jnp.dot(w3_ref[...], Xb[b], preferred_element_type=f32)
            for b in bs]
    qkv = [qkvf[b].astype(bf16) for b in bs]
    qT = [qkv[b][0:_C, :] for b in bs]
    kT = [qkv[b][_C:2 * _C, :] for b in bs]
    vT = [qkv[b][2 * _C:3 * _C, :] for b in bs]

    dotT = lambda a, c: lax.dot_general(a, c, (((1,), (1,)), ((), ())),
                                        preferred_element_type=f32)
    Sraw = [dotT(qT[b], kT[b]) for b in bs]
    qksq = [jnp.sum(qkvf[b][0:2 * _C, :] * qkvf[b][0:2 * _C, :],
                    axis=1, keepdims=True) for b in bs]
    rq = [1.0 / jnp.maximum(jnp.sqrt(qksq[b][0:_C]), 1e-12) for b in bs]
    rk = [1.0 / jnp.maximum(jnp.sqrt(qksq[b][_C:2 * _C]), 1e-12).reshape(1, _C)
          for b in bs]

    r1 = lax.broadcasted_iota(jnp.int32, (_C, _C), 0) // _DH
    r2 = lax.broadcasted_iota(jnp.int32, (_C, _C), 1) // _DH
    blk = r1 == r2
    A = []
    for b in bs:
        S = jnp.where(blk, Sraw[b] * rq[b] * rk[b], -1e30)
        S = S - jnp.max(S, axis=1, keepdims=True)
        E = jnp.exp(S)
        A.append(E / jnp.sum(E, axis=1, keepdims=True))

    WgA = [jnp.dot(wg_ref[...], A[b].astype(bf16),
                   preferred_element_type=f32).astype(bf16) for b in bs]
    CAt = [jnp.dot(WgA[b], vT[b], preferred_element_type=f32) for b in bs]
    attn = [X[b] + CAt[b] + bg_ref[...] for b in bs]

    n = lax.broadcasted_iota(jnp.int32, (1, _N), 1)
    ww = (n >> 4) & 15
    dd = n & 15
    for b in bs:
        vstk_ref[b, :, 0:_HPAD] = jnp.zeros((9 * _C, _HPAD), bf16)
        vstk_ref[b, :, _HPAD + _N:_VW] = jnp.zeros((9 * _C, _HPAD), bf16)
        vstk_ref[b, 4 * _C:5 * _C, _HPAD:_HPAD + _N] = attn[b].astype(bf16)

    for b in bs:
        j = 0
        for db in (-1, 0, 1):
            for dc in (-1, 0, 1):
                if db == 0 and dc == 0:
                    j += 1
                    continue
                s = db * 16 + dc
                sl = vstk_ref[b, 4 * _C:5 * _C, _HPAD + s:_HPAD + s + _N]
                cond = None
                for d, v in ((db, ww), (dc, dd)):
                    if d == -1:
                        c = v >= 1
                    elif d == 1:
                        c = v <= 14
                    else:
                        continue
                    cond = c if cond is None else (cond & c)
                if cond is not None:
                    sl = jnp.where(cond, sl, jnp.zeros((), bf16))
                vstk_ref[b, j * _C:(j + 1) * _C, _HPAD:_HPAD + _N] = sl
                j += 1

    _T = _N // 2
    acc = [[None, None] for _ in bs]
    s1 = [None, None]
    s2 = [None, None]
    for b in bs:
        for t in range(2):
            c0 = t * _T
            a = (jnp.dot(wstk_ref[0], vstk_ref[b, :, c0:c0 + _T],
                         preferred_element_type=f32)
                 + jnp.dot(wstk_ref[1], vstk_ref[b, :, 256 + c0:256 + c0 + _T],
                           preferred_element_type=f32)
                 + jnp.dot(wstk_ref[2], vstk_ref[b, :, 512 + c0:512 + c0 + _T],
                           preferred_element_type=f32))
            acc[b][t] = a
            p1 = jnp.sum(a)
            p2 = jnp.sum(a * a)
            s1[b] = p1 if t == 0 else s1[b] + p1
            s2[b] = p2 if t == 0 else s2[b] + p2

    cnt = float(_C * _N)
    for b in bs:
        mean = s1[b] / cnt
        var = s2[b] / cnt - mean * mean
        rstd = lax.rsqrt(var + 1e-5)
        for t in range(2):
            c0 = t * _T
            gn = (acc[b][t] - mean) * rstd * gnw_ref[...] + gnb_ref[...]
            o = gn + attn[b][:, c0:c0 + _T]
            out_ref[b, :, c0:c0 + _T] = jnp.where(o >= 0, o, 0.01 * o)


def kernel(x, Wqkvv, Wout1, bout1, Wout2, bout2, gamma1, conv_w, gn_w, gn_b,
           index_sample):
    B, C, H, W, D = x.shape
    N = H * W * D
    x_cn = x.reshape(B, C, N)
    w3 = Wqkvv[:3 * C, :].astype(jnp.bfloat16)
    wg = (gamma1[:, None] * Wout2).astype(jnp.bfloat16)
    bg = (gamma1 * bout2).reshape(C, 1)
    wstk = conv_w.astype(jnp.bfloat16).transpose(2, 0, 3, 4, 1).reshape(3, C, 9 * C)

    full = lambda *shape: pl.BlockSpec(shape, lambda: (0,) * len(shape))
    out = pl.pallas_call(
        _fused_block,
        grid=(),
        in_specs=[
            full(B, C, N),
            full(3 * C, C),
            full(C, C),
            full(C, 1),
            full(3, C, 9 * C),
            full(C, 1),
            full(C, 1),
        ],
        out_specs=full(B, C, N),
        out_shape=jax.ShapeDtypeStruct((B, C, N), jnp.float32),
        scratch_shapes=[pltpu.VMEM((B, 9 * C, _VW), jnp.bfloat16)],
    )(x_cn, w3, wg, bg, wstk, gn_w.reshape(C, 1), gn_b.reshape(C, 1))
    return out.reshape(B, C, H, W, D)


# speedup vs baseline: 1.2281x; 1.0076x over previous
import jax
import jax.numpy as jnp
from jax import lax
from jax.experimental import pallas as pl
from jax.experimental.pallas import tpu as pltpu

_B = 2
_C = 128
_DH = 32
_HWD = 16
_N = _HWD * _HWD * _HWD
_PAD = 32
_PW = _N + 2 * _PAD
_HPAD = 256
_VW = _N + 2 * _HPAD


def _fused_block(x_ref, w3_ref, wg_ref, bg_ref, wstk_ref, gnw_ref,
                 gnb_ref, out_ref, vstk_ref):
    f32 = jnp.float32
    bf16 = jnp.bfloat16
    bs = range(_B)
    X = [x_ref[b] for b in bs]
    Xb = [X[b].astype(bf16) for b in bs]

    qkvf = [jnp.dot(w3_ref[...], Xb[b], preferred_element_type=f32)
            for b in bs]
    qkv = [qkvf[b].astype(bf16) for b in bs]
    qT = [qkv[b][0:_C, :] for b in bs]
    kT = [qkv[b][_C:2 * _C, :] for b in bs]
    vT = [qkv[b][2 * _C:3 * _C, :] for b in bs]

    dotT = lambda a, c: lax.dot_general(a, c, (((1,), (1,)), ((), ())),
                                        preferred_element_type=f32)
    Sraw = [dotT(qT[b], kT[b]) for b in bs]
    qksq = [jnp.sum(qkvf[b][0:2 * _C, :] * qkvf[b][0:2 * _C, :],
                    axis=1, keepdims=True) for b in bs]
    rq = [1.0 / jnp.maximum(jnp.sqrt(qksq[b][0:_C]), 1e-12) for b in bs]
    rk = [1.0 / jnp.maximum(jnp.sqrt(qksq[b][_C:2 * _C]), 1e-12).reshape(1, _C)
          for b in bs]

    r1 = lax.broadcasted_iota(jnp.int32, (_C, _C), 0) // _DH
    r2 = lax.broadcasted_iota(jnp.int32, (_C, _C), 1) // _DH
    blk = r1 == r2
    A = []
    for b in bs:
        S = jnp.where(blk, Sraw[b] * rq[b] * rk[b], -1e30)
        S = S - jnp.max(S, axis=1, keepdims=True)
        E = jnp.exp(S)
        A.append(E / jnp.sum(E, axis=1, keepdims=True))

    WgA = [jnp.dot(wg_ref[...], A[b].astype(bf16),
                   preferred_element_type=f32).astype(bf16) for b in bs]
    CAt = [jnp.dot(WgA[b], vT[b], preferred_element_type=f32) for b in bs]
    attn = [X[b] + CAt[b] + bg_ref[...] for b in bs]

    n = lax.broadcasted_iota(jnp.int32, (1, _N), 1)
    ww = (n >> 4) & 15
    dd = n & 15
    for b in bs:
        vstk_ref[b, :, 0:_HPAD] = jnp.zeros((9 * _C, _HPAD), bf16)
        vstk_ref[b, :, _HPAD + _N:_VW] = jnp.zeros((9 * _C, _HPAD), bf16)
        vstk_ref[b, 4 * _C:5 * _C, _HPAD:_HPAD + _N] = attn[b].astype(bf16)

    for b in bs:
        j = 0
        for db in (-1, 0, 1):
            for dc in (-1, 0, 1):
                if db == 0 and dc == 0:
                    j += 1
                    continue
                s = db * 16 + dc
                sl = vstk_ref[b, 4 * _C:5 * _C, _HPAD + s:_HPAD + s + _N]
                cond = None
                for d, v in ((db, ww), (dc, dd)):
                    if d == -1:
                        c = v >= 1
                    elif d == 1:
                        c = v <= 14
                    else:
                        continue
                    cond = c if cond is None else (cond & c)
                if cond is not None:
                    sl = jnp.where(cond, sl, jnp.zeros((), bf16))
                vstk_ref[b, j * _C:(j + 1) * _C, _HPAD:_HPAD + _N] = sl
                j += 1

    _T = _N // 4
    acc = [[None] * 4 for _ in bs]
    s1 = [None, None]
    s2 = [None, None]
    for b in bs:
        for t in range(4):
            c0 = t * _T
            a = (jnp.dot(wstk_ref[0], vstk_ref[b, :, c0:c0 + _T],
                         preferred_element_type=f32)
                 + jnp.dot(wstk_ref[1], vstk_ref[b, :, 256 + c0:256 + c0 + _T],
                           preferred_element_type=f32)
                 + jnp.dot(wstk_ref[2], vstk_ref[b, :, 512 + c0:512 + c0 + _T],
                           preferred_element_type=f32))
            acc[b][t] = a
            p1 = jnp.sum(a)
            p2 = jnp.sum(a * a)
            s1[b] = p1 if t == 0 else s1[b] + p1
            s2[b] = p2 if t == 0 else s2[b] + p2

    cnt = float(_C * _N)
    for b in bs:
        mean = s1[b] / cnt
        var = s2[b] / cnt - mean * mean
        rstd = lax.rsqrt(var + 1e-5)
        for t in range(4):
            c0 = t * _T
            gn = (acc[b][t] - mean) * rstd * gnw_ref[...] + gnb_ref[...]
            o = gn + attn[b][:, c0:c0 + _T]
            out_ref[b, :, c0:c0 + _T] = jnp.where(o >= 0, o, 0.01 * o)


def kernel(x, Wqkvv, Wout1, bout1, Wout2, bout2, gamma1, conv_w, gn_w, gn_b,
           index_sample):
    B, C, H, W, D = x.shape
    N = H * W * D
    x_cn = x.reshape(B, C, N)
    w3 = Wqkvv[:3 * C, :].astype(jnp.bfloat16)
    wg = (gamma1[:, None] * Wout2).astype(jnp.bfloat16)
    bg = (gamma1 * bout2).reshape(C, 1)
    wstk = conv_w.astype(jnp.bfloat16).transpose(2, 0, 3, 4, 1).reshape(3, C, 9 * C)

    full = lambda *shape: pl.BlockSpec(shape, lambda: (0,) * len(shape))
    out = pl.pallas_call(
        _fused_block,
        grid=(),
        in_specs=[
            full(B, C, N),
            full(3 * C, C),
            full(C, C),
            full(C, 1),
            full(3, C, 9 * C),
            full(C, 1),
            full(C, 1),
        ],
        out_specs=full(B, C, N),
        out_shape=jax.ShapeDtypeStruct((B, C, N), jnp.float32),
        scratch_shapes=[pltpu.VMEM((B, 9 * C, _VW), jnp.bfloat16)],
    )(x_cn, w3, wg, bg, wstk, gn_w.reshape(C, 1), gn_b.reshape(C, 1))
    return out.reshape(B, C, H, W, D)
